# Initial kernel scaffold; baseline (speedup 1.0000x reference)
import jax
import jax.numpy as jnp
from jax.experimental import pallas as pl


def kernel(emb_u1, emb_i1, emb_u2, emb_i2, fc1_w, fc1_b, fc2_w, fc2_b, fc3_w, fc3_b, fc4_w, fc4_b, vals1, vals2, users_cnt, items_cnt, users, edge_index1, edge_index2):
    raise NotImplementedError("write your pallas kernel here")


# speedup vs baseline: 5.2993x; 5.2993x over previous
import functools

import jax
import jax.numpy as jnp
from jax import lax
from jax.experimental import pallas as pl
from jax.experimental.pallas import tpu as pltpu
from jax.experimental.pallas import tpu_sc as plsc

NUM_USERS = 50000
NUM_ITEMS = 50000
N_NODES = NUM_USERS + NUM_ITEMS
D = 32
N_LAYERS = 3
E = 1600000
LAM1 = 0.9
LAM2 = 0.9
BATCH = 1024

HALF = 50176
NP = 2 * HALF
PAD_OFF = HALF - NUM_USERS

NSC = 2
NTS = 16
EPT = 100352
E_PAD = NTS * EPT
BLK = 512
NBLK = EPT // BLK
CHUNK = 128
NCH = BLK // CHUNK
RPT = HALF // NTS
RCH = 112
NRCH = RPT // RCH

_mesh = plsc.VectorSubcoreMesh(core_axis_name="c", subcore_axis_name="s")


def _sc_layer_body(row_h, col_h, val_h, tab_h, accin_h, newtab_h, accout_h,
                   col_v, row_v, val_v, idx_v, src_v, zb_v, ab_v, acc_sh, sem):
    c = lax.axis_index("c")
    s = lax.axis_index("s")

    def zrow(i, _):
        zb_v[i, pl.ds(0, 16)] = jnp.zeros((16,), jnp.float32)
        zb_v[i, pl.ds(16, 16)] = jnp.zeros((16,), jnp.float32)
        return _
    lax.fori_loop(0, RCH, zrow, None)
    for k in range(NRCH):
        pltpu.sync_copy(zb_v, acc_sh.at[pl.ds(s * RPT + k * RCH, RCH)])
    plsc.subcore_barrier()

    ebase = s * EPT
    coff = c * NUM_USERS
    trash = NUM_USERS + s

    def blk(b, _):
        base = ebase + b * BLK
        pltpu.sync_copy(col_h.at[pl.ds(base, BLK)], col_v)
        pltpu.sync_copy(row_h.at[pl.ds(base, BLK)], row_v)
        pltpu.sync_copy(val_h.at[pl.ds(base, BLK)], val_v)

        def tf(g, _):
            o = g * 16
            cg = col_v[pl.ds(o, 16)]
            cg = cg + jnp.where(cg >= NUM_USERS, PAD_OFF, 0)
            col_v[pl.ds(o, 16)] = cg
            rg = row_v[pl.ds(o, 16)] - coff
            bad = (rg < 0) | (rg >= NUM_USERS)
            k = o // CHUNK
            idx_v[k, pl.ds(o - k * CHUNK, 16)] = jnp.where(bad, trash, rg)
            return _
        lax.fori_loop(0, BLK // 16, tf, None)

        descs = [
            pltpu.async_copy(tab_h.at[col_v.at[pl.ds(k * CHUNK, CHUNK)]],
                             src_v.at[pl.ds(k * CHUNK, CHUNK)], sem)
            for k in range(NCH)
        ]
        for d in descs:
            d.wait()

        def grp(g, _):
            o = g * 16
            vv = val_v[pl.ds(o, 16)]
            for e in range(16):
                eg = o + e
                bc = lax.gather(
                    vv, jnp.full((16, 1), e, jnp.int32),
                    lax.GatherDimensionNumbers(offset_dims=(),
                                               collapsed_slice_dims=(0,),
                                               start_index_map=(0,)),
                    (1,), mode=lax.GatherScatterMode.PROMISE_IN_BOUNDS)
                src_v[eg, pl.ds(0, 16)] = src_v[eg, pl.ds(0, 16)] * bc
                src_v[eg, pl.ds(16, 16)] = src_v[eg, pl.ds(16, 16)] * bc
            return _
        lax.fori_loop(0, BLK // 16, grp, None)

        for k in range(NCH):
            pltpu.sync_copy(src_v.at[pl.ds(k * CHUNK, CHUNK)],
                            acc_sh.at[idx_v.at[k]], add=True)
        return _
    lax.fori_loop(0, NBLK, blk, None)
    plsc.subcore_barrier()

    for k in range(NRCH):
        r0 = s * RPT + k * RCH
        g0 = c * HALF + r0
        pltpu.sync_copy(acc_sh.at[pl.ds(r0, RCH)], zb_v)
        pltpu.sync_copy(accin_h.at[pl.ds(g0, RCH)], ab_v)

        def acc_row(i, _):
            a0 = zb_v[i, pl.ds(0, 16)]
            a1 = zb_v[i, pl.ds(16, 16)]
            ab_v[i, pl.ds(0, 16)] = ab_v[i, pl.ds(0, 16)] + a0
            ab_v[i, pl.ds(16, 16)] = ab_v[i, pl.ds(16, 16)] + a1
            return _
        lax.fori_loop(0, RCH, acc_row, None)
        pltpu.sync_copy(zb_v, newtab_h.at[pl.ds(g0, RCH)])
        pltpu.sync_copy(ab_v, accout_h.at[pl.ds(g0, RCH)])


@functools.partial(
    pl.kernel,
    out_type=(jax.ShapeDtypeStruct((NP, D), jnp.float32),
              jax.ShapeDtypeStruct((NP, D), jnp.float32)),
    mesh=_mesh,
    scratch_types=[
        pltpu.VMEM((BLK,), jnp.int32),
        pltpu.VMEM((BLK,), jnp.int32),
        pltpu.VMEM((BLK,), jnp.float32),
        pltpu.VMEM((NCH, CHUNK), jnp.int32),
        pltpu.VMEM((BLK, D), jnp.float32),
        pltpu.VMEM((RCH, D), jnp.float32),
        pltpu.VMEM((RCH, D), jnp.float32),
        pltpu.VMEM_SHARED((HALF, D), jnp.float32),
        pltpu.SemaphoreType.DMA,
    ],
    compiler_params=pltpu.CompilerParams(use_tc_tiling_on_sc=False),
)
def _sc_layer(row_h, col_h, val_h, tab_h, accin_h, newtab_h, accout_h,
              *scratch):
    _sc_layer_body(row_h, col_h, val_h, tab_h, accin_h, newtab_h, accout_h,
                   *scratch)


UPT = BATCH // (NSC * NTS)


@functools.partial(
    pl.kernel,
    out_type=(jax.ShapeDtypeStruct((BATCH, D), jnp.float32),
              jax.ShapeDtypeStruct((BATCH, D), jnp.float32),
              jax.ShapeDtypeStruct((BATCH,), jnp.float32)),
    mesh=_mesh,
    scratch_types=[
        pltpu.VMEM((UPT,), jnp.int32),
        pltpu.VMEM((UPT, D), jnp.float32),
        pltpu.VMEM((UPT, D), jnp.float32),
        pltpu.VMEM((UPT,), jnp.float32),
        pltpu.SemaphoreType.DMA,
    ],
    compiler_params=pltpu.CompilerParams(use_tc_tiling_on_sc=False),
)
def _sc_batch_gather(users_h, acc2_h, acc1_h, ucnt_h, ue2_h, ue1_h, ucb_h,
                     uidx_v, u2_v, u1_v, uc_v, sem):
    c = lax.axis_index("c")
    s = lax.axis_index("s")
    wid = s * NSC + c
    base = wid * UPT
    pltpu.sync_copy(users_h.at[pl.ds(base, UPT)], uidx_v)
    pltpu.async_copy(acc2_h.at[uidx_v], u2_v, sem).wait()
    pltpu.async_copy(acc1_h.at[uidx_v], u1_v, sem).wait()
    pltpu.async_copy(ucnt_h.at[uidx_v], uc_v, sem).wait()
    pltpu.sync_copy(u2_v, ue2_h.at[pl.ds(base, UPT)])
    pltpu.sync_copy(u1_v, ue1_h.at[pl.ds(base, UPT)])
    pltpu.sync_copy(uc_v, ucb_h.at[pl.ds(base, UPT)])


IBLK = 512
IGRID = 98


def _tc_rating_body(ue2, ue1, ucb, i2, i1, icnt,
                    f1w, f1b, f2w, f2b, f3w, f3b, f4w, f4b, out):
    au = ue2[...] * 0.25
    au2 = ue1[...] * 0.25
    z1 = (jax.lax.dot_general(au, f1w[...], (((1,), (0,)), ((), ())),
                              preferred_element_type=jnp.float32) + f1b[...]
          + jax.lax.dot_general(au2, f2w[...], (((1,), (0,)), ((), ())),
                                preferred_element_type=jnp.float32) + f2b[...])
    w1 = jax.nn.sigmoid(z1)
    w1 = ucb[...] * (1.0 - LAM1) + w1 * LAM1
    u = au * w1 + au2 * (1.0 - w1)

    ai = i2[...] * 0.25
    ai2 = i1[...] * 0.25
    z2 = (jax.lax.dot_general(ai, f3w[...], (((1,), (0,)), ((), ())),
                              preferred_element_type=jnp.float32) + f3b[...]
          + jax.lax.dot_general(ai2, f4w[...], (((1,), (0,)), ((), ())),
                                preferred_element_type=jnp.float32) + f4b[...])
    w2 = jax.nn.sigmoid(z2)
    w2 = icnt[...] * (1.0 - LAM2) + w2 * LAM2
    fi = ai * w2 + ai2 * (1.0 - w2)

    r = jax.lax.dot_general(u, fi, (((1,), (1,)), ((), ())),
                            preferred_element_type=jnp.float32)
    out[...] = jax.nn.sigmoid(r)


def _tc_rating(ue2, ue1, ucb, acc2, acc1, items_cnt,
               f1w, f1b, f2w, f2b, f3w, f3b, f4w, f4b):
    full = lambda j: (0, 0)
    return pl.pallas_call(
        _tc_rating_body,
        grid=(IGRID,),
        in_specs=[
            pl.BlockSpec((BATCH, D), full),
            pl.BlockSpec((BATCH, D), full),
            pl.BlockSpec((BATCH, 1), full),
            pl.BlockSpec((IBLK, D), lambda j: (IGRID + j, 0)),
            pl.BlockSpec((IBLK, D), lambda j: (IGRID + j, 0)),
            pl.BlockSpec((IBLK, 1), lambda j: (j, 0)),
            pl.BlockSpec((D, 1), full),
            pl.BlockSpec((1, 1), full),
            pl.BlockSpec((D, 1), full),
            pl.BlockSpec((1, 1), full),
            pl.BlockSpec((D, 1), full),
            pl.BlockSpec((1, 1), full),
            pl.BlockSpec((D, 1), full),
            pl.BlockSpec((1, 1), full),
        ],
        out_specs=pl.BlockSpec((BATCH, IBLK), lambda j: (0, j)),
        out_shape=jax.ShapeDtypeStruct((BATCH, NUM_ITEMS), jnp.float32),
    )(ue2, ue1, ucb, acc2, acc1, items_cnt,
      f1w, f1b, f2w, f2b, f3w, f3b, f4w, f4b)


def _pad_table(emb_u, emb_i):
    zpad = jnp.zeros((PAD_OFF, D), jnp.float32)
    return jnp.concatenate([emb_u, zpad, emb_i, zpad], axis=0)


def _propagate_sc(edge_index, vals, x_pad):
    pad = E_PAD - E
    rows = jnp.pad(edge_index[0], (0, pad), constant_values=-1)
    cols = jnp.pad(edge_index[1], (0, pad), constant_values=0)
    v = jnp.pad(vals, (0, pad), constant_values=0.0)
    tab = x_pad
    acc = x_pad
    for _ in range(N_LAYERS):
        tab, acc = _sc_layer(rows, cols, v, tab, acc)
    return acc


def kernel(emb_u1, emb_i1, emb_u2, emb_i2, fc1_w, fc1_b, fc2_w, fc2_b,
           fc3_w, fc3_b, fc4_w, fc4_b, vals1, vals2, users_cnt, items_cnt,
           users, edge_index1, edge_index2):
    x2 = _pad_table(emb_u2, emb_i2)
    x1 = _pad_table(emb_u1, emb_i1)
    acc2 = _propagate_sc(edge_index2, vals2, x2)
    acc1 = _propagate_sc(edge_index1, vals1, x1)

    ue2, ue1, ucb = _sc_batch_gather(users, acc2, acc1,
                                     users_cnt.reshape(NUM_USERS))
    rating = _tc_rating(ue2, ue1, ucb.reshape(BATCH, 1), acc2, acc1,
                        items_cnt,
                        fc1_w, fc1_b.reshape(1, 1), fc2_w, fc2_b.reshape(1, 1),
                        fc3_w, fc3_b.reshape(1, 1), fc4_w, fc4_b.reshape(1, 1))
    return rating


# speedup vs baseline: 8.1350x; 1.5351x over previous
import functools

import jax
import jax.numpy as jnp
from jax import lax
from jax.experimental import pallas as pl
from jax.experimental.pallas import tpu as pltpu
from jax.experimental.pallas import tpu_sc as plsc

NUM_USERS = 50000
NUM_ITEMS = 50000
N_NODES = NUM_USERS + NUM_ITEMS
D = 32
N_LAYERS = 3
E = 1600000
LAM1 = 0.9
LAM2 = 0.9
BATCH = 1024

HALF = 50176
NP = 2 * HALF
PAD_OFF = HALF - NUM_USERS

NSC = 2
NTS = 16
EPT = 100352
E_PAD = NTS * EPT
BLK = 256
NBLK = EPT // BLK
CHUNK = 128
NCH = BLK // CHUNK
RPT = HALF // NTS
RCH = 112
NRCH = RPT // RCH

_mesh = plsc.VectorSubcoreMesh(core_axis_name="c", subcore_axis_name="s")


def _sc_layer_body(row_h, col_h, val_h, tab_h, accin_h, newtab_h, accout_h,
                   col_v, row_v, val_v, idx_v, src_v, zb_v, ab_v, acc_sh,
                   sem, sem_s, sem_g, sem_w):
    c = lax.axis_index("c")
    s = lax.axis_index("s")

    def zrow(i, _):
        zb_v[i, pl.ds(0, 16)] = jnp.zeros((16,), jnp.float32)
        zb_v[i, pl.ds(16, 16)] = jnp.zeros((16,), jnp.float32)
        return _
    lax.fori_loop(0, RCH, zrow, None)
    for k in range(NRCH):
        pltpu.sync_copy(zb_v, acc_sh.at[pl.ds(s * RPT + k * RCH, RCH)])
    plsc.subcore_barrier()

    ebase = s * EPT
    coff = c * NUM_USERS
    trash = NUM_USERS + s

    def issue_smalls(p, base):
        pltpu.async_copy(col_h.at[pl.ds(base, BLK)], col_v.at[p], sem_s)
        pltpu.async_copy(row_h.at[pl.ds(base, BLK)], row_v.at[p], sem_s)
        pltpu.async_copy(val_h.at[pl.ds(base, BLK)], val_v.at[p], sem_s)

    def wait_smalls(p):
        pltpu.make_async_copy(col_h.at[pl.ds(0, BLK)], col_v.at[p],
                              sem_s).wait()
        pltpu.make_async_copy(row_h.at[pl.ds(0, BLK)], row_v.at[p],
                              sem_s).wait()
        pltpu.make_async_copy(val_h.at[pl.ds(0, BLK)], val_v.at[p],
                              sem_s).wait()

    def tf_block(p):
        def tf(g, _):
            o = g * 16
            cg = col_v[p, pl.ds(o, 16)]
            cg = cg + jnp.where(cg >= NUM_USERS, PAD_OFF, 0)
            col_v[p, pl.ds(o, 16)] = cg
            rg = row_v[p, pl.ds(o, 16)] - coff
            bad = (rg < 0) | (rg >= NUM_USERS)
            k = o // CHUNK
            idx_v[p, k, pl.ds(o - k * CHUNK, 16)] = jnp.where(bad, trash, rg)
            return _
        lax.fori_loop(0, BLK // 16, tf, None)

    def issue_gather(p):
        for k in range(NCH):
            pltpu.async_copy(
                tab_h.at[col_v.at[p, pl.ds(k * CHUNK, CHUNK)]],
                src_v.at[p, pl.ds(k * CHUNK, CHUNK)], sem_g)

    def wait_gather(p):
        for k in range(NCH):
            pltpu.make_async_copy(
                tab_h.at[col_v.at[p, pl.ds(k * CHUNK, CHUNK)]],
                src_v.at[p, pl.ds(k * CHUNK, CHUNK)], sem_g).wait()

    def issue_scatter(p):
        for k in range(NCH):
            pltpu.async_copy(src_v.at[p, pl.ds(k * CHUNK, CHUNK)],
                             acc_sh.at[idx_v.at[p, k]], sem_w, add=True)

    def wait_scatter(p):
        for k in range(NCH):
            pltpu.make_async_copy(src_v.at[p, pl.ds(k * CHUNK, CHUNK)],
                                  acc_sh.at[idx_v.at[p, k]], sem_w).wait()

    def compute(p):
        def grp(g, _):
            o = g * 16
            vv = val_v[p, pl.ds(o, 16)]
            for e in range(16):
                eg = o + e
                bc = lax.gather(
                    vv, jnp.full((16, 1), e, jnp.int32),
                    lax.GatherDimensionNumbers(offset_dims=(),
                                               collapsed_slice_dims=(0,),
                                               start_index_map=(0,)),
                    (1,), mode=lax.GatherScatterMode.PROMISE_IN_BOUNDS)
                src_v[p, eg, pl.ds(0, 16)] = src_v[p, eg, pl.ds(0, 16)] * bc
                src_v[p, eg, pl.ds(16, 16)] = src_v[p, eg, pl.ds(16, 16)] * bc
            return _
        lax.fori_loop(0, BLK // 16, grp, None)

    issue_smalls(0, ebase)
    wait_smalls(0)
    tf_block(0)
    issue_gather(0)
    issue_smalls(1, ebase + BLK)

    def blk(b, _):
        p = lax.rem(b, 2)
        q = 1 - p
        wait_gather(p)

        @pl.when(b < NBLK - 1)
        def _():
            wait_smalls(q)
            tf_block(q)

            @pl.when(b > 0)
            def _():
                wait_scatter(q)
            issue_gather(q)

        compute(p)
        issue_scatter(p)

        @pl.when(b < NBLK - 2)
        def _():
            issue_smalls(p, ebase + (b + 2) * BLK)
        return _
    lax.fori_loop(0, NBLK, blk, None)
    wait_scatter(0)
    wait_scatter(1)
    plsc.subcore_barrier()

    for k in range(NRCH):
        r0 = s * RPT + k * RCH
        g0 = c * HALF + r0
        pltpu.sync_copy(acc_sh.at[pl.ds(r0, RCH)], zb_v)
        pltpu.sync_copy(accin_h.at[pl.ds(g0, RCH)], ab_v)

        def acc_row(i, _):
            a0 = zb_v[i, pl.ds(0, 16)]
            a1 = zb_v[i, pl.ds(16, 16)]
            ab_v[i, pl.ds(0, 16)] = ab_v[i, pl.ds(0, 16)] + a0
            ab_v[i, pl.ds(16, 16)] = ab_v[i, pl.ds(16, 16)] + a1
            return _
        lax.fori_loop(0, RCH, acc_row, None)
        pltpu.sync_copy(zb_v, newtab_h.at[pl.ds(g0, RCH)])
        pltpu.sync_copy(ab_v, accout_h.at[pl.ds(g0, RCH)])


@functools.partial(
    pl.kernel,
    out_type=(jax.ShapeDtypeStruct((NP, D), jnp.float32),
              jax.ShapeDtypeStruct((NP, D), jnp.float32)),
    mesh=_mesh,
    scratch_types=[
        pltpu.VMEM((2, BLK), jnp.int32),
        pltpu.VMEM((2, BLK), jnp.int32),
        pltpu.VMEM((2, BLK), jnp.float32),
        pltpu.VMEM((2, NCH, CHUNK), jnp.int32),
        pltpu.VMEM((2, BLK, D), jnp.float32),
        pltpu.VMEM((RCH, D), jnp.float32),
        pltpu.VMEM((RCH, D), jnp.float32),
        pltpu.VMEM_SHARED((HALF, D), jnp.float32),
        pltpu.SemaphoreType.DMA,
        pltpu.SemaphoreType.DMA,
        pltpu.SemaphoreType.DMA,
        pltpu.SemaphoreType.DMA,
    ],
    compiler_params=pltpu.CompilerParams(use_tc_tiling_on_sc=False),
)
def _sc_layer(row_h, col_h, val_h, tab_h, accin_h, newtab_h, accout_h,
              *scratch):
    _sc_layer_body(row_h, col_h, val_h, tab_h, accin_h, newtab_h, accout_h,
                   *scratch)


UPT = BATCH // (NSC * NTS)


@functools.partial(
    pl.kernel,
    out_type=(jax.ShapeDtypeStruct((BATCH, D), jnp.float32),
              jax.ShapeDtypeStruct((BATCH, D), jnp.float32),
              jax.ShapeDtypeStruct((BATCH,), jnp.float32)),
    mesh=_mesh,
    scratch_types=[
        pltpu.VMEM((UPT,), jnp.int32),
        pltpu.VMEM((UPT, D), jnp.float32),
        pltpu.VMEM((UPT, D), jnp.float32),
        pltpu.VMEM((UPT,), jnp.float32),
        pltpu.SemaphoreType.DMA,
    ],
    compiler_params=pltpu.CompilerParams(use_tc_tiling_on_sc=False),
)
def _sc_batch_gather(users_h, acc2_h, acc1_h, ucnt_h, ue2_h, ue1_h, ucb_h,
                     uidx_v, u2_v, u1_v, uc_v, sem):
    c = lax.axis_index("c")
    s = lax.axis_index("s")
    wid = s * NSC + c
    base = wid * UPT
    pltpu.sync_copy(users_h.at[pl.ds(base, UPT)], uidx_v)
    pltpu.async_copy(acc2_h.at[uidx_v], u2_v, sem).wait()
    pltpu.async_copy(acc1_h.at[uidx_v], u1_v, sem).wait()
    pltpu.async_copy(ucnt_h.at[uidx_v], uc_v, sem).wait()
    pltpu.sync_copy(u2_v, ue2_h.at[pl.ds(base, UPT)])
    pltpu.sync_copy(u1_v, ue1_h.at[pl.ds(base, UPT)])
    pltpu.sync_copy(uc_v, ucb_h.at[pl.ds(base, UPT)])


IBLK = 512
IGRID = 98


def _tc_rating_body(ue2, ue1, ucb, i2, i1, icnt,
                    f1w, f1b, f2w, f2b, f3w, f3b, f4w, f4b, out):
    au = ue2[...] * 0.25
    au2 = ue1[...] * 0.25
    z1 = (jax.lax.dot_general(au, f1w[...], (((1,), (0,)), ((), ())),
                              preferred_element_type=jnp.float32) + f1b[...]
          + jax.lax.dot_general(au2, f2w[...], (((1,), (0,)), ((), ())),
                                preferred_element_type=jnp.float32) + f2b[...])
    w1 = jax.nn.sigmoid(z1)
    w1 = ucb[...] * (1.0 - LAM1) + w1 * LAM1
    u = au * w1 + au2 * (1.0 - w1)

    ai = i2[...] * 0.25
    ai2 = i1[...] * 0.25
    z2 = (jax.lax.dot_general(ai, f3w[...], (((1,), (0,)), ((), ())),
                              preferred_element_type=jnp.float32) + f3b[...]
          + jax.lax.dot_general(ai2, f4w[...], (((1,), (0,)), ((), ())),
                                preferred_element_type=jnp.float32) + f4b[...])
    w2 = jax.nn.sigmoid(z2)
    w2 = icnt[...] * (1.0 - LAM2) + w2 * LAM2
    fi = ai * w2 + ai2 * (1.0 - w2)

    r = jax.lax.dot_general(u, fi, (((1,), (1,)), ((), ())),
                            preferred_element_type=jnp.float32)
    out[...] = jax.nn.sigmoid(r)


def _tc_rating(ue2, ue1, ucb, acc2, acc1, items_cnt,
               f1w, f1b, f2w, f2b, f3w, f3b, f4w, f4b):
    full = lambda j: (0, 0)
    return pl.pallas_call(
        _tc_rating_body,
        grid=(IGRID,),
        in_specs=[
            pl.BlockSpec((BATCH, D), full),
            pl.BlockSpec((BATCH, D), full),
            pl.BlockSpec((BATCH, 1), full),
            pl.BlockSpec((IBLK, D), lambda j: (IGRID + j, 0)),
            pl.BlockSpec((IBLK, D), lambda j: (IGRID + j, 0)),
            pl.BlockSpec((IBLK, 1), lambda j: (j, 0)),
            pl.BlockSpec((D, 1), full),
            pl.BlockSpec((1, 1), full),
            pl.BlockSpec((D, 1), full),
            pl.BlockSpec((1, 1), full),
            pl.BlockSpec((D, 1), full),
            pl.BlockSpec((1, 1), full),
            pl.BlockSpec((D, 1), full),
            pl.BlockSpec((1, 1), full),
        ],
        out_specs=pl.BlockSpec((BATCH, IBLK), lambda j: (0, j)),
        out_shape=jax.ShapeDtypeStruct((BATCH, NUM_ITEMS), jnp.float32),
    )(ue2, ue1, ucb, acc2, acc1, items_cnt,
      f1w, f1b, f2w, f2b, f3w, f3b, f4w, f4b)


def _pad_table(emb_u, emb_i):
    zpad = jnp.zeros((PAD_OFF, D), jnp.float32)
    return jnp.concatenate([emb_u, zpad, emb_i, zpad], axis=0)


def _propagate_sc(edge_index, vals, x_pad):
    pad = E_PAD - E
    rows = jnp.pad(edge_index[0], (0, pad), constant_values=-1)
    cols = jnp.pad(edge_index[1], (0, pad), constant_values=0)
    v = jnp.pad(vals, (0, pad), constant_values=0.0)
    tab = x_pad
    acc = x_pad
    for _ in range(N_LAYERS):
        tab, acc = _sc_layer(rows, cols, v, tab, acc)
    return acc


def kernel(emb_u1, emb_i1, emb_u2, emb_i2, fc1_w, fc1_b, fc2_w, fc2_b,
           fc3_w, fc3_b, fc4_w, fc4_b, vals1, vals2, users_cnt, items_cnt,
           users, edge_index1, edge_index2):
    x2 = _pad_table(emb_u2, emb_i2)
    x1 = _pad_table(emb_u1, emb_i1)
    acc2 = _propagate_sc(edge_index2, vals2, x2)
    acc1 = _propagate_sc(edge_index1, vals1, x1)

    ue2, ue1, ucb = _sc_batch_gather(users, acc2, acc1,
                                     users_cnt.reshape(NUM_USERS))
    rating = _tc_rating(ue2, ue1, ucb.reshape(BATCH, 1), acc2, acc1,
                        items_cnt,
                        fc1_w, fc1_b.reshape(1, 1), fc2_w, fc2_b.reshape(1, 1),
                        fc3_w, fc3_b.reshape(1, 1), fc4_w, fc4_b.reshape(1, 1))
    return rating


# speedup vs baseline: 10.4679x; 1.2868x over previous
import functools

import jax
import jax.numpy as jnp
from jax import lax
from jax.experimental import pallas as pl
from jax.experimental.pallas import tpu as pltpu
from jax.experimental.pallas import tpu_sc as plsc

NUM_USERS = 50000
NUM_ITEMS = 50000
N_NODES = NUM_USERS + NUM_ITEMS
D = 32
N_LAYERS = 3
E = 1600000
LAM1 = 0.9
LAM2 = 0.9
BATCH = 1024

HALF = 50176
NP = 2 * HALF
PAD_OFF = HALF - NUM_USERS

NSC = 2
NTS = 16
NW = NSC * NTS
CAP = 50176
E_PAD = NW * CAP
SEG = NW * CAP
BLK = 256
NBR = CAP // BLK
CHUNK = 128
NCH = BLK // CHUNK
RPT = HALF // NTS
RCH = 112
NRCH = RPT // RCH
STG = 544

_mesh = plsc.VectorSubcoreMesh(core_axis_name="c", subcore_axis_name="s")
_I32 = jnp.int32


def _route_body(col_h, row_h, val_h, segc_h, segr_h, segv_h, cnt_h,
                colb, rowb, valb, sc0, sr0, sv0, sc1, sr1, sv1, cntv):
    c = lax.axis_index("c")
    s = lax.axis_index("s")
    j = s * NSC + c
    ebase = j * CAP
    trash = NUM_USERS + j // 2

    def blk_fn(b, carry):
        f0, f1, bc0, bc1 = carry
        base = ebase + b * BLK
        pltpu.sync_copy(col_h.at[pl.ds(base, BLK)], colb)
        pltpu.sync_copy(row_h.at[pl.ds(base, BLK)], rowb)
        pltpu.sync_copy(val_h.at[pl.ds(base, BLK)], valb)

        lane = lax.iota(_I32, 16)

        def grp(g, cr):
            f0, f1 = cr
            o = g * 16
            cg = colb[pl.ds(o, 16)]
            rg = rowb[pl.ds(o, 16)]
            vg = valb[pl.ds(o, 16)]
            pc = cg + jnp.where(cg >= NUM_USERS, PAD_OFF, 0)
            m0 = (rg >= 0) & (rg < NUM_USERS)
            m1 = rg >= NUM_USERS
            cs0 = plsc.cumsum(m0.astype(_I32))
            pos0 = f0 + cs0 - 1
            plsc.store_scatter(sc0, [pos0], pc, mask=m0)
            plsc.store_scatter(sr0, [pos0], rg, mask=m0)
            plsc.store_scatter(sv0, [pos0], vg, mask=m0)
            cs1 = plsc.cumsum(m1.astype(_I32))
            pos1 = f1 + cs1 - 1
            plsc.store_scatter(sc1, [pos1], pc, mask=m1)
            plsc.store_scatter(sr1, [pos1], rg - NUM_USERS, mask=m1)
            plsc.store_scatter(sv1, [pos1], vg, mask=m1)
            n0 = jnp.max(cs0)
            n1 = jnp.max(cs1)
            return (f0 + n0, f1 + n1)
        f0, f1 = lax.fori_loop(0, BLK // 16, grp, (f0, f1))

        def flush(stc, str_, stv, seg_off, f, bc, do):
            @pl.when(do)
            def _():
                pltpu.sync_copy(stc.at[pl.ds(0, BLK)],
                                segc_h.at[pl.ds(seg_off + bc * BLK, BLK)])
                pltpu.sync_copy(str_.at[pl.ds(0, BLK)],
                                segr_h.at[pl.ds(seg_off + bc * BLK, BLK)])
                pltpu.sync_copy(stv.at[pl.ds(0, BLK)],
                                segv_h.at[pl.ds(seg_off + bc * BLK, BLK)])

                def mv(g, _):
                    o = g * 16
                    stc[pl.ds(o, 16)] = stc[pl.ds(BLK + o, 16)]
                    str_[pl.ds(o, 16)] = str_[pl.ds(BLK + o, 16)]
                    stv[pl.ds(o, 16)] = stv[pl.ds(BLK + o, 16)]
                    return _
                lax.fori_loop(0, BLK // 16, mv, None)
            return (jnp.where(do, f - BLK, f), jnp.where(do, bc + 1, bc))

        f0, bc0 = flush(sc0, sr0, sv0, (j * CAP), f0, bc0, f0 >= BLK)
        f1, bc1 = flush(sc1, sr1, sv1, (SEG + j * CAP), f1, bc1, f1 >= BLK)
        return (f0, f1, bc0, bc1)

    z = jnp.int32(0)
    f0, f1, bc0, bc1 = lax.fori_loop(0, NBR, blk_fn, (z, z, z, z))

    def tail(stc, str_, stv, seg_off, f, bc):
        def pg(g, _):
            o = g * 16
            lane = o + lax.iota(jnp.int32, 16)
            keep = lane < f
            stc[pl.ds(o, 16)] = jnp.where(keep, stc[pl.ds(o, 16)], 0)
            str_[pl.ds(o, 16)] = jnp.where(keep, str_[pl.ds(o, 16)], trash)
            stv[pl.ds(o, 16)] = jnp.where(keep, stv[pl.ds(o, 16)], 0.0)
            return _
        lax.fori_loop(0, BLK // 16, pg, None)

        @pl.when(f > 0)
        def _():
            pltpu.sync_copy(stc.at[pl.ds(0, BLK)],
                            segc_h.at[pl.ds(seg_off + bc * BLK, BLK)])
            pltpu.sync_copy(str_.at[pl.ds(0, BLK)],
                            segr_h.at[pl.ds(seg_off + bc * BLK, BLK)])
            pltpu.sync_copy(stv.at[pl.ds(0, BLK)],
                            segv_h.at[pl.ds(seg_off + bc * BLK, BLK)])
        return jnp.where(f > 0, bc + 1, bc)

    bc0 = tail(sc0, sr0, sv0, (j * CAP), f0, bc0)
    bc1 = tail(sc1, sr1, sv1, (SEG + j * CAP), f1, bc1)

    cntv[pl.ds(0, 16)] = jnp.full((16,), bc0, _I32)
    pltpu.sync_copy(cntv, cnt_h.at[0, j])
    cntv[pl.ds(0, 16)] = jnp.full((16,), bc1, _I32)
    pltpu.sync_copy(cntv, cnt_h.at[1, j])


@functools.partial(
    pl.kernel,
    out_type=(jax.ShapeDtypeStruct((2 * SEG,), _I32),
              jax.ShapeDtypeStruct((2 * SEG,), _I32),
              jax.ShapeDtypeStruct((2 * SEG,), jnp.float32),
              jax.ShapeDtypeStruct((2, NW, 16), _I32)),
    mesh=_mesh,
    scratch_types=[
        pltpu.VMEM((BLK,), _I32),
        pltpu.VMEM((BLK,), _I32),
        pltpu.VMEM((BLK,), jnp.float32),
        pltpu.VMEM((STG,), _I32),
        pltpu.VMEM((STG,), _I32),
        pltpu.VMEM((STG,), jnp.float32),
        pltpu.VMEM((STG,), _I32),
        pltpu.VMEM((STG,), _I32),
        pltpu.VMEM((STG,), jnp.float32),
        pltpu.VMEM((16,), _I32),
    ],
    compiler_params=pltpu.CompilerParams(use_tc_tiling_on_sc=False,
                                         needs_layout_passes=False),
)
def _sc_route(col_h, row_h, val_h, *rest):
    _route_body(col_h, row_h, val_h, *rest)


def _sc_layer_body(segc_h, segr_h, segv_h, cnt_h, tab_h, accin_h,
                   newtab_h, accout_h,
                   col_v, val_v, idx_v, src_v, zb_v, ab_v, cnt_v, acc_sh,
                   sem_s, sem_g, sem_w):
    c = lax.axis_index("c")
    s = lax.axis_index("s")

    def zrow(i, _):
        zb_v[i, pl.ds(0, 16)] = jnp.zeros((16,), jnp.float32)
        zb_v[i, pl.ds(16, 16)] = jnp.zeros((16,), jnp.float32)
        return _
    lax.fori_loop(0, RCH, zrow, None)
    for k in range(NRCH):
        pltpu.sync_copy(zb_v, acc_sh.at[pl.ds(s * RPT + k * RCH, RCH)])
    plsc.subcore_barrier()

    pltpu.sync_copy(cnt_h.at[c, 2 * s], cnt_v)
    nb0 = jnp.max(cnt_v[...])
    pltpu.sync_copy(cnt_h.at[c, 2 * s + 1], cnt_v)
    nb1 = jnp.max(cnt_v[...])
    nb = nb0 + nb1
    hoff = c * SEG

    def base(b):
        return hoff + jnp.where(
            b < nb0, (2 * s) * CAP + b * BLK,
            (2 * s + 1) * CAP + (b - nb0) * BLK)

    def issue_smalls(p, base_):
        pltpu.async_copy(segc_h.at[pl.ds(base_, BLK)], col_v.at[p], sem_s)
        for k in range(NCH):
            pltpu.async_copy(segr_h.at[pl.ds(base_ + k * CHUNK, CHUNK)],
                             idx_v.at[p, k], sem_s)
        pltpu.async_copy(segv_h.at[pl.ds(base_, BLK)], val_v.at[p], sem_s)

    def wait_smalls(p):
        pltpu.make_async_copy(segc_h.at[pl.ds(0, BLK)], col_v.at[p],
                              sem_s).wait()
        for k in range(NCH):
            pltpu.make_async_copy(segr_h.at[pl.ds(0, CHUNK)],
                                  idx_v.at[p, k], sem_s).wait()
        pltpu.make_async_copy(segv_h.at[pl.ds(0, BLK)], val_v.at[p],
                              sem_s).wait()

    def issue_gather(p):
        for k in range(NCH):
            pltpu.async_copy(
                tab_h.at[col_v.at[p, pl.ds(k * CHUNK, CHUNK)]],
                src_v.at[p, pl.ds(k * CHUNK, CHUNK)], sem_g)

    def wait_gather(p):
        for k in range(NCH):
            pltpu.make_async_copy(
                tab_h.at[col_v.at[p, pl.ds(k * CHUNK, CHUNK)]],
                src_v.at[p, pl.ds(k * CHUNK, CHUNK)], sem_g).wait()

    def issue_scatter(p):
        for k in range(NCH):
            pltpu.async_copy(src_v.at[p, pl.ds(k * CHUNK, CHUNK)],
                             acc_sh.at[idx_v.at[p, k]], sem_w, add=True)

    def wait_scatter(p):
        for k in range(NCH):
            pltpu.make_async_copy(src_v.at[p, pl.ds(k * CHUNK, CHUNK)],
                                  acc_sh.at[idx_v.at[p, k]], sem_w).wait()

    def compute(p):
        def grp(g, _):
            o = g * 16
            vv = val_v[p, pl.ds(o, 16)]
            for e in range(16):
                eg = o + e
                bc = lax.gather(
                    vv, jnp.full((16, 1), e, _I32),
                    lax.GatherDimensionNumbers(offset_dims=(),
                                               collapsed_slice_dims=(0,),
                                               start_index_map=(0,)),
                    (1,), mode=lax.GatherScatterMode.PROMISE_IN_BOUNDS)
                src_v[p, eg, pl.ds(0, 16)] = src_v[p, eg, pl.ds(0, 16)] * bc
                src_v[p, eg, pl.ds(16, 16)] = src_v[p, eg, pl.ds(16, 16)] * bc
            return _
        lax.fori_loop(0, BLK // 16, grp, None)

    @pl.when(nb > 0)
    def _():
        issue_smalls(0, base(0))
        wait_smalls(0)
        issue_gather(0)

    @pl.when(nb > 1)
    def _():
        issue_smalls(1, base(1))

    def blk(b, _):
        p = lax.rem(b, 2)
        q = 1 - p
        wait_gather(p)

        @pl.when(b < nb - 1)
        def _():
            wait_smalls(q)

            @pl.when(b > 0)
            def _():
                wait_scatter(q)
            issue_gather(q)

        compute(p)
        issue_scatter(p)

        @pl.when(b < nb - 2)
        def _():
            issue_smalls(p, base(b + 2))
        return _
    lax.fori_loop(0, nb, blk, None)

    @pl.when(nb > 1)
    def _():
        wait_scatter(lax.rem(nb, 2))

    @pl.when(nb > 0)
    def _():
        wait_scatter(lax.rem(nb + 1, 2))
    plsc.subcore_barrier()

    for k in range(NRCH):
        r0 = s * RPT + k * RCH
        g0 = c * HALF + r0
        pltpu.sync_copy(acc_sh.at[pl.ds(r0, RCH)], zb_v)
        pltpu.sync_copy(accin_h.at[pl.ds(g0, RCH)], ab_v)

        def acc_row(i, _):
            ab_v[i, pl.ds(0, 16)] = ab_v[i, pl.ds(0, 16)] + zb_v[i, pl.ds(0, 16)]
            ab_v[i, pl.ds(16, 16)] = (ab_v[i, pl.ds(16, 16)]
                                      + zb_v[i, pl.ds(16, 16)])
            return _
        lax.fori_loop(0, RCH, acc_row, None)
        pltpu.sync_copy(zb_v, newtab_h.at[pl.ds(g0, RCH)])
        pltpu.sync_copy(ab_v, accout_h.at[pl.ds(g0, RCH)])


@functools.partial(
    pl.kernel,
    out_type=(jax.ShapeDtypeStruct((NP, D), jnp.float32),
              jax.ShapeDtypeStruct((NP, D), jnp.float32)),
    mesh=_mesh,
    scratch_types=[
        pltpu.VMEM((2, BLK), _I32),
        pltpu.VMEM((2, BLK), jnp.float32),
        pltpu.VMEM((2, NCH, CHUNK), _I32),
        pltpu.VMEM((2, BLK, D), jnp.float32),
        pltpu.VMEM((RCH, D), jnp.float32),
        pltpu.VMEM((RCH, D), jnp.float32),
        pltpu.VMEM((16,), _I32),
        pltpu.VMEM_SHARED((HALF, D), jnp.float32),
        pltpu.SemaphoreType.DMA,
        pltpu.SemaphoreType.DMA,
        pltpu.SemaphoreType.DMA,
    ],
    compiler_params=pltpu.CompilerParams(use_tc_tiling_on_sc=False,
                                         needs_layout_passes=False),
)
def _sc_layer(segc_h, segr_h, segv_h, cnt_h, tab_h, accin_h, *rest):
    _sc_layer_body(segc_h, segr_h, segv_h, cnt_h, tab_h, accin_h, *rest)


UPT = BATCH // NW


@functools.partial(
    pl.kernel,
    out_type=(jax.ShapeDtypeStruct((BATCH, D), jnp.float32),
              jax.ShapeDtypeStruct((BATCH, D), jnp.float32),
              jax.ShapeDtypeStruct((BATCH,), jnp.float32)),
    mesh=_mesh,
    scratch_types=[
        pltpu.VMEM((UPT,), _I32),
        pltpu.VMEM((UPT, D), jnp.float32),
        pltpu.VMEM((UPT, D), jnp.float32),
        pltpu.VMEM((UPT,), jnp.float32),
        pltpu.SemaphoreType.DMA,
    ],
    compiler_params=pltpu.CompilerParams(use_tc_tiling_on_sc=False),
)
def _sc_batch_gather(users_h, acc2_h, acc1_h, ucnt_h, ue2_h, ue1_h, ucb_h,
                     uidx_v, u2_v, u1_v, uc_v, sem):
    c = lax.axis_index("c")
    s = lax.axis_index("s")
    wid = s * NSC + c
    base = wid * UPT
    pltpu.sync_copy(users_h.at[pl.ds(base, UPT)], uidx_v)
    pltpu.async_copy(acc2_h.at[uidx_v], u2_v, sem).wait()
    pltpu.async_copy(acc1_h.at[uidx_v], u1_v, sem).wait()
    pltpu.async_copy(ucnt_h.at[uidx_v], uc_v, sem).wait()
    pltpu.sync_copy(u2_v, ue2_h.at[pl.ds(base, UPT)])
    pltpu.sync_copy(u1_v, ue1_h.at[pl.ds(base, UPT)])
    pltpu.sync_copy(uc_v, ucb_h.at[pl.ds(base, UPT)])


IBLK = 512
IGRID = 98


def _tc_rating_body(ue2, ue1, ucb, i2, i1, icnt,
                    f1w, f1b, f2w, f2b, f3w, f3b, f4w, f4b, out):
    au = ue2[...] * 0.25
    au2 = ue1[...] * 0.25
    z1 = (jax.lax.dot_general(au, f1w[...], (((1,), (0,)), ((), ())),
                              preferred_element_type=jnp.float32) + f1b[...]
          + jax.lax.dot_general(au2, f2w[...], (((1,), (0,)), ((), ())),
                                preferred_element_type=jnp.float32) + f2b[...])
    w1 = jax.nn.sigmoid(z1)
    w1 = ucb[...] * (1.0 - LAM1) + w1 * LAM1
    u = au * w1 + au2 * (1.0 - w1)

    ai = i2[...] * 0.25
    ai2 = i1[...] * 0.25
    z2 = (jax.lax.dot_general(ai, f3w[...], (((1,), (0,)), ((), ())),
                              preferred_element_type=jnp.float32) + f3b[...]
          + jax.lax.dot_general(ai2, f4w[...], (((1,), (0,)), ((), ())),
                                preferred_element_type=jnp.float32) + f4b[...])
    w2 = jax.nn.sigmoid(z2)
    w2 = icnt[...] * (1.0 - LAM2) + w2 * LAM2
    fi = ai * w2 + ai2 * (1.0 - w2)

    r = jax.lax.dot_general(u, fi, (((1,), (1,)), ((), ())),
                            preferred_element_type=jnp.float32)
    out[...] = jax.nn.sigmoid(r)


def _tc_rating(ue2, ue1, ucb, acc2, acc1, items_cnt,
               f1w, f1b, f2w, f2b, f3w, f3b, f4w, f4b):
    full = lambda j: (0, 0)
    return pl.pallas_call(
        _tc_rating_body,
        grid=(IGRID,),
        in_specs=[
            pl.BlockSpec((BATCH, D), full),
            pl.BlockSpec((BATCH, D), full),
            pl.BlockSpec((BATCH, 1), full),
            pl.BlockSpec((IBLK, D), lambda j: (IGRID + j, 0)),
            pl.BlockSpec((IBLK, D), lambda j: (IGRID + j, 0)),
            pl.BlockSpec((IBLK, 1), lambda j: (j, 0)),
            pl.BlockSpec((D, 1), full),
            pl.BlockSpec((1, 1), full),
            pl.BlockSpec((D, 1), full),
            pl.BlockSpec((1, 1), full),
            pl.BlockSpec((D, 1), full),
            pl.BlockSpec((1, 1), full),
            pl.BlockSpec((D, 1), full),
            pl.BlockSpec((1, 1), full),
        ],
        out_specs=pl.BlockSpec((BATCH, IBLK), lambda j: (0, j)),
        out_shape=jax.ShapeDtypeStruct((BATCH, NUM_ITEMS), jnp.float32),
    )(ue2, ue1, ucb, acc2, acc1, items_cnt,
      f1w, f1b, f2w, f2b, f3w, f3b, f4w, f4b)


def _pad_table(emb_u, emb_i):
    zpad = jnp.zeros((PAD_OFF, D), jnp.float32)
    return jnp.concatenate([emb_u, zpad, emb_i, zpad], axis=0)


def _propagate_sc(edge_index, vals, x_pad):
    pad = E_PAD - E
    rows = jnp.pad(edge_index[0], (0, pad), constant_values=-1)
    cols = jnp.pad(edge_index[1], (0, pad), constant_values=0)
    v = jnp.pad(vals, (0, pad), constant_values=0.0)
    segc, segr, segv, cnt = _sc_route(cols, rows, v)
    tab = x_pad
    acc = x_pad
    for _ in range(N_LAYERS):
        tab, acc = _sc_layer(segc, segr, segv, cnt, tab, acc)
    return acc


def kernel(emb_u1, emb_i1, emb_u2, emb_i2, fc1_w, fc1_b, fc2_w, fc2_b,
           fc3_w, fc3_b, fc4_w, fc4_b, vals1, vals2, users_cnt, items_cnt,
           users, edge_index1, edge_index2):
    x2 = _pad_table(emb_u2, emb_i2)
    x1 = _pad_table(emb_u1, emb_i1)
    acc2 = _propagate_sc(edge_index2, vals2, x2)
    acc1 = _propagate_sc(edge_index1, vals1, x1)

    ue2, ue1, ucb = _sc_batch_gather(users, acc2, acc1,
                                     users_cnt.reshape(NUM_USERS))
    rating = _tc_rating(ue2, ue1, ucb.reshape(BATCH, 1), acc2, acc1,
                        items_cnt,
                        fc1_w, fc1_b.reshape(1, 1), fc2_w, fc2_b.reshape(1, 1),
                        fc3_w, fc3_b.reshape(1, 1), fc4_w, fc4_b.reshape(1, 1))
    return rating


# speedup vs baseline: 12.5220x; 1.1962x over previous
import functools

import jax
import jax.numpy as jnp
from jax import lax
from jax.experimental import pallas as pl
from jax.experimental.pallas import tpu as pltpu
from jax.experimental.pallas import tpu_sc as plsc

NUM_USERS = 50000
NUM_ITEMS = 50000
N_NODES = NUM_USERS + NUM_ITEMS
D = 32
N_LAYERS = 3
E = 1600000
LAM1 = 0.9
LAM2 = 0.9
BATCH = 1024

HALF = 50176
NP = 2 * HALF
PAD_OFF = HALF - NUM_USERS

NSC = 2
NTS = 16
NW = NSC * NTS
CAP = 50176
E_PAD = NW * CAP
SEG = NW * CAP
BLK = 256
NBR = CAP // BLK
CHUNK = 128
NCH = BLK // CHUNK
RPT = HALF // NTS
RCH = 112
NRCH = RPT // RCH
STG = 544

_mesh = plsc.VectorSubcoreMesh(core_axis_name="c", subcore_axis_name="s")
_I32 = jnp.int32


def _route_body(col_h, row_h, val_h, segc_h, segr_h, segv_h, cnt_h,
                colb, rowb, valb, sc0, sr0, sv0, sc1, sr1, sv1, cntv, sem_r):
    c = lax.axis_index("c")
    s = lax.axis_index("s")
    j = s * NSC + c
    ebase = j * CAP
    trash = NUM_USERS + j // 2

    def issue_in(p, base):
        pltpu.async_copy(col_h.at[pl.ds(base, BLK)], colb.at[p], sem_r)
        pltpu.async_copy(row_h.at[pl.ds(base, BLK)], rowb.at[p], sem_r)
        pltpu.async_copy(val_h.at[pl.ds(base, BLK)], valb.at[p], sem_r)

    def wait_in(p):
        pltpu.make_async_copy(col_h.at[pl.ds(0, BLK)], colb.at[p],
                              sem_r).wait()
        pltpu.make_async_copy(row_h.at[pl.ds(0, BLK)], rowb.at[p],
                              sem_r).wait()
        pltpu.make_async_copy(val_h.at[pl.ds(0, BLK)], valb.at[p],
                              sem_r).wait()

    issue_in(0, ebase)

    def blk_fn(b, carry):
        f0, f1, bc0, bc1 = carry
        p = lax.rem(b, 2)
        wait_in(p)

        @pl.when(b < NBR - 1)
        def _():
            issue_in(1 - p, ebase + (b + 1) * BLK)

        def grp(g, cr):
            f0, f1 = cr
            o = g * 16
            cg = colb[p, pl.ds(o, 16)]
            rg = rowb[p, pl.ds(o, 16)]
            vg = valb[p, pl.ds(o, 16)]
            pc = cg + jnp.where(cg >= NUM_USERS, PAD_OFF, 0)
            m0 = (rg >= 0) & (rg < NUM_USERS)
            m1 = rg >= NUM_USERS
            cs0 = plsc.cumsum(m0.astype(_I32))
            pos0 = f0 + cs0 - 1
            plsc.store_scatter(sc0, [pos0], pc, mask=m0)
            plsc.store_scatter(sr0, [pos0], rg, mask=m0)
            plsc.store_scatter(sv0, [pos0], vg, mask=m0)
            cs1 = plsc.cumsum(m1.astype(_I32))
            pos1 = f1 + cs1 - 1
            plsc.store_scatter(sc1, [pos1], pc, mask=m1)
            plsc.store_scatter(sr1, [pos1], rg - NUM_USERS, mask=m1)
            plsc.store_scatter(sv1, [pos1], vg, mask=m1)
            n0 = jnp.max(cs0)
            n1 = jnp.max(cs1)
            return (f0 + n0, f1 + n1)
        f0, f1 = lax.fori_loop(0, BLK // 16, grp, (f0, f1))

        def flush(stc, str_, stv, seg_off, f, bc, do):
            @pl.when(do)
            def _():
                pltpu.sync_copy(stc.at[pl.ds(0, BLK)],
                                segc_h.at[pl.ds(seg_off + bc * BLK, BLK)])
                pltpu.sync_copy(str_.at[pl.ds(0, BLK)],
                                segr_h.at[pl.ds(seg_off + bc * BLK, BLK)])
                pltpu.sync_copy(stv.at[pl.ds(0, BLK)],
                                segv_h.at[pl.ds(seg_off + bc * BLK, BLK)])

                def mv(g, _):
                    o = g * 16
                    stc[pl.ds(o, 16)] = stc[pl.ds(BLK + o, 16)]
                    str_[pl.ds(o, 16)] = str_[pl.ds(BLK + o, 16)]
                    stv[pl.ds(o, 16)] = stv[pl.ds(BLK + o, 16)]
                    return _
                lax.fori_loop(0, BLK // 16, mv, None)
            return (jnp.where(do, f - BLK, f), jnp.where(do, bc + 1, bc))

        f0, bc0 = flush(sc0, sr0, sv0, (j * CAP), f0, bc0, f0 >= BLK)
        f1, bc1 = flush(sc1, sr1, sv1, (SEG + j * CAP), f1, bc1, f1 >= BLK)
        return (f0, f1, bc0, bc1)

    z = jnp.int32(0)
    f0, f1, bc0, bc1 = lax.fori_loop(0, NBR, blk_fn, (z, z, z, z))

    def tail(stc, str_, stv, seg_off, f, bc):
        def pg(g, _):
            o = g * 16
            lane = o + lax.iota(jnp.int32, 16)
            keep = lane < f
            stc[pl.ds(o, 16)] = jnp.where(keep, stc[pl.ds(o, 16)], 0)
            str_[pl.ds(o, 16)] = jnp.where(keep, str_[pl.ds(o, 16)], trash)
            stv[pl.ds(o, 16)] = jnp.where(keep, stv[pl.ds(o, 16)], 0.0)
            return _
        lax.fori_loop(0, BLK // 16, pg, None)

        @pl.when(f > 0)
        def _():
            pltpu.sync_copy(stc.at[pl.ds(0, BLK)],
                            segc_h.at[pl.ds(seg_off + bc * BLK, BLK)])
            pltpu.sync_copy(str_.at[pl.ds(0, BLK)],
                            segr_h.at[pl.ds(seg_off + bc * BLK, BLK)])
            pltpu.sync_copy(stv.at[pl.ds(0, BLK)],
                            segv_h.at[pl.ds(seg_off + bc * BLK, BLK)])
        return jnp.where(f > 0, bc + 1, bc)

    bc0 = tail(sc0, sr0, sv0, (j * CAP), f0, bc0)
    bc1 = tail(sc1, sr1, sv1, (SEG + j * CAP), f1, bc1)

    cntv[pl.ds(0, 16)] = jnp.full((16,), bc0, _I32)
    pltpu.sync_copy(cntv, cnt_h.at[0, j])
    cntv[pl.ds(0, 16)] = jnp.full((16,), bc1, _I32)
    pltpu.sync_copy(cntv, cnt_h.at[1, j])


@functools.partial(
    pl.kernel,
    out_type=(jax.ShapeDtypeStruct((2 * SEG,), _I32),
              jax.ShapeDtypeStruct((2 * SEG,), _I32),
              jax.ShapeDtypeStruct((2 * SEG,), jnp.float32),
              jax.ShapeDtypeStruct((2, NW, 16), _I32)),
    mesh=_mesh,
    scratch_types=[
        pltpu.VMEM((2, BLK), _I32),
        pltpu.VMEM((2, BLK), _I32),
        pltpu.VMEM((2, BLK), jnp.float32),
        pltpu.VMEM((STG,), _I32),
        pltpu.VMEM((STG,), _I32),
        pltpu.VMEM((STG,), jnp.float32),
        pltpu.VMEM((STG,), _I32),
        pltpu.VMEM((STG,), _I32),
        pltpu.VMEM((STG,), jnp.float32),
        pltpu.VMEM((16,), _I32),
        pltpu.SemaphoreType.DMA,
    ],
    compiler_params=pltpu.CompilerParams(use_tc_tiling_on_sc=False,
                                         needs_layout_passes=False),
)
def _sc_route(col_h, row_h, val_h, *rest):
    _route_body(col_h, row_h, val_h, *rest)


def _sc_layer_body(segc_h, segr_h, segv_h, cnt_h, tab_h, accin_h,
                   newtab_h, accout_h,
                   col_v, val_v, idx_v, src_v, zb_v, ab_v, cnt_v, acc_sh,
                   sem_s, sem_g, sem_w):
    c = lax.axis_index("c")
    s = lax.axis_index("s")

    def zrow(i, _):
        zb_v[i, pl.ds(0, 16)] = jnp.zeros((16,), jnp.float32)
        zb_v[i, pl.ds(16, 16)] = jnp.zeros((16,), jnp.float32)
        return _
    lax.fori_loop(0, RCH, zrow, None)
    for k in range(NRCH):
        pltpu.sync_copy(zb_v, acc_sh.at[pl.ds(s * RPT + k * RCH, RCH)])
    plsc.subcore_barrier()

    pltpu.sync_copy(cnt_h.at[c, 2 * s], cnt_v)
    nb0 = jnp.max(cnt_v[...])
    pltpu.sync_copy(cnt_h.at[c, 2 * s + 1], cnt_v)
    nb1 = jnp.max(cnt_v[...])
    nb = nb0 + nb1
    hoff = c * SEG

    def base(b):
        return hoff + jnp.where(
            b < nb0, (2 * s) * CAP + b * BLK,
            (2 * s + 1) * CAP + (b - nb0) * BLK)

    def issue_smalls(p, base_):
        pltpu.async_copy(segc_h.at[pl.ds(base_, BLK)], col_v.at[p], sem_s)
        for k in range(NCH):
            pltpu.async_copy(segr_h.at[pl.ds(base_ + k * CHUNK, CHUNK)],
                             idx_v.at[p, k], sem_s)
        pltpu.async_copy(segv_h.at[pl.ds(base_, BLK)], val_v.at[p], sem_s)

    def wait_smalls(p):
        pltpu.make_async_copy(segc_h.at[pl.ds(0, BLK)], col_v.at[p],
                              sem_s).wait()
        for k in range(NCH):
            pltpu.make_async_copy(segr_h.at[pl.ds(0, CHUNK)],
                                  idx_v.at[p, k], sem_s).wait()
        pltpu.make_async_copy(segv_h.at[pl.ds(0, BLK)], val_v.at[p],
                              sem_s).wait()

    def issue_gather(p):
        for k in range(NCH):
            pltpu.async_copy(
                tab_h.at[col_v.at[p, pl.ds(k * CHUNK, CHUNK)]],
                src_v.at[p, pl.ds(k * CHUNK, CHUNK)], sem_g)

    def wait_gather(p):
        for k in range(NCH):
            pltpu.make_async_copy(
                tab_h.at[col_v.at[p, pl.ds(k * CHUNK, CHUNK)]],
                src_v.at[p, pl.ds(k * CHUNK, CHUNK)], sem_g).wait()

    def issue_scatter(p):
        for k in range(NCH):
            pltpu.async_copy(src_v.at[p, pl.ds(k * CHUNK, CHUNK)],
                             acc_sh.at[idx_v.at[p, k]], sem_w, add=True)

    def wait_scatter(p):
        for k in range(NCH):
            pltpu.make_async_copy(src_v.at[p, pl.ds(k * CHUNK, CHUNK)],
                                  acc_sh.at[idx_v.at[p, k]], sem_w).wait()

    def compute(p):
        def grp(g, _):
            o = g * 16
            vv = val_v[p, pl.ds(o, 16)]
            for e in range(16):
                eg = o + e
                bc = lax.gather(
                    vv, jnp.full((16, 1), e, _I32),
                    lax.GatherDimensionNumbers(offset_dims=(),
                                               collapsed_slice_dims=(0,),
                                               start_index_map=(0,)),
                    (1,), mode=lax.GatherScatterMode.PROMISE_IN_BOUNDS)
                src_v[p, eg, pl.ds(0, 16)] = src_v[p, eg, pl.ds(0, 16)] * bc
                src_v[p, eg, pl.ds(16, 16)] = src_v[p, eg, pl.ds(16, 16)] * bc
            return _
        lax.fori_loop(0, BLK // 16, grp, None)

    @pl.when(nb > 0)
    def _():
        issue_smalls(0, base(0))
        wait_smalls(0)
        issue_gather(0)

    @pl.when(nb > 1)
    def _():
        issue_smalls(1, base(1))

    def blk(b, _):
        p = lax.rem(b, 2)
        q = 1 - p
        wait_gather(p)

        @pl.when(b < nb - 1)
        def _():
            wait_smalls(q)

            @pl.when(b > 0)
            def _():
                wait_scatter(q)
            issue_gather(q)

        compute(p)
        issue_scatter(p)

        @pl.when(b < nb - 2)
        def _():
            issue_smalls(p, base(b + 2))
        return _
    lax.fori_loop(0, nb, blk, None)

    @pl.when(nb > 1)
    def _():
        wait_scatter(lax.rem(nb, 2))

    @pl.when(nb > 0)
    def _():
        wait_scatter(lax.rem(nb + 1, 2))
    plsc.subcore_barrier()

    for k in range(NRCH):
        r0 = s * RPT + k * RCH
        g0 = c * HALF + r0
        pltpu.sync_copy(acc_sh.at[pl.ds(r0, RCH)], zb_v)
        pltpu.sync_copy(accin_h.at[pl.ds(g0, RCH)], ab_v)

        def acc_row(i, _):
            ab_v[i, pl.ds(0, 16)] = ab_v[i, pl.ds(0, 16)] + zb_v[i, pl.ds(0, 16)]
            ab_v[i, pl.ds(16, 16)] = (ab_v[i, pl.ds(16, 16)]
                                      + zb_v[i, pl.ds(16, 16)])
            return _
        lax.fori_loop(0, RCH, acc_row, None)
        pltpu.sync_copy(zb_v, newtab_h.at[pl.ds(g0, RCH)])
        pltpu.sync_copy(ab_v, accout_h.at[pl.ds(g0, RCH)])


@functools.partial(
    pl.kernel,
    out_type=(jax.ShapeDtypeStruct((NP, D), jnp.float32),
              jax.ShapeDtypeStruct((NP, D), jnp.float32)),
    mesh=_mesh,
    scratch_types=[
        pltpu.VMEM((2, BLK), _I32),
        pltpu.VMEM((2, BLK), jnp.float32),
        pltpu.VMEM((2, NCH, CHUNK), _I32),
        pltpu.VMEM((2, BLK, D), jnp.float32),
        pltpu.VMEM((RCH, D), jnp.float32),
        pltpu.VMEM((RCH, D), jnp.float32),
        pltpu.VMEM((16,), _I32),
        pltpu.VMEM_SHARED((HALF, D), jnp.float32),
        pltpu.SemaphoreType.DMA,
        pltpu.SemaphoreType.DMA,
        pltpu.SemaphoreType.DMA,
    ],
    compiler_params=pltpu.CompilerParams(use_tc_tiling_on_sc=False,
                                         needs_layout_passes=False),
)
def _sc_layer(segc_h, segr_h, segv_h, cnt_h, tab_h, accin_h, *rest):
    _sc_layer_body(segc_h, segr_h, segv_h, cnt_h, tab_h, accin_h, *rest)


UPT = BATCH // NW


@functools.partial(
    pl.kernel,
    out_type=(jax.ShapeDtypeStruct((BATCH, D), jnp.float32),
              jax.ShapeDtypeStruct((BATCH, D), jnp.float32),
              jax.ShapeDtypeStruct((BATCH,), jnp.float32)),
    mesh=_mesh,
    scratch_types=[
        pltpu.VMEM((UPT,), _I32),
        pltpu.VMEM((UPT, D), jnp.float32),
        pltpu.VMEM((UPT, D), jnp.float32),
        pltpu.VMEM((UPT,), jnp.float32),
        pltpu.SemaphoreType.DMA,
    ],
    compiler_params=pltpu.CompilerParams(use_tc_tiling_on_sc=False),
)
def _sc_batch_gather(users_h, acc2_h, acc1_h, ucnt_h, ue2_h, ue1_h, ucb_h,
                     uidx_v, u2_v, u1_v, uc_v, sem):
    c = lax.axis_index("c")
    s = lax.axis_index("s")
    wid = s * NSC + c
    base = wid * UPT
    pltpu.sync_copy(users_h.at[pl.ds(base, UPT)], uidx_v)
    pltpu.async_copy(acc2_h.at[uidx_v], u2_v, sem).wait()
    pltpu.async_copy(acc1_h.at[uidx_v], u1_v, sem).wait()
    pltpu.async_copy(ucnt_h.at[uidx_v], uc_v, sem).wait()
    pltpu.sync_copy(u2_v, ue2_h.at[pl.ds(base, UPT)])
    pltpu.sync_copy(u1_v, ue1_h.at[pl.ds(base, UPT)])
    pltpu.sync_copy(uc_v, ucb_h.at[pl.ds(base, UPT)])


IBLK = 512
IGRID = 98


def _tc_rating_body(ue2, ue1, ucb, i2, i1, icnt,
                    f1w, f1b, f2w, f2b, f3w, f3b, f4w, f4b, out):
    au = ue2[...] * 0.25
    au2 = ue1[...] * 0.25
    z1 = (jax.lax.dot_general(au, f1w[...], (((1,), (0,)), ((), ())),
                              preferred_element_type=jnp.float32) + f1b[...]
          + jax.lax.dot_general(au2, f2w[...], (((1,), (0,)), ((), ())),
                                preferred_element_type=jnp.float32) + f2b[...])
    w1 = jax.nn.sigmoid(z1)
    w1 = ucb[...] * (1.0 - LAM1) + w1 * LAM1
    u = au * w1 + au2 * (1.0 - w1)

    ai = i2[...] * 0.25
    ai2 = i1[...] * 0.25
    z2 = (jax.lax.dot_general(ai, f3w[...], (((1,), (0,)), ((), ())),
                              preferred_element_type=jnp.float32) + f3b[...]
          + jax.lax.dot_general(ai2, f4w[...], (((1,), (0,)), ((), ())),
                                preferred_element_type=jnp.float32) + f4b[...])
    w2 = jax.nn.sigmoid(z2)
    w2 = icnt[...] * (1.0 - LAM2) + w2 * LAM2
    fi = ai * w2 + ai2 * (1.0 - w2)

    r = jax.lax.dot_general(u, fi, (((1,), (1,)), ((), ())),
                            preferred_element_type=jnp.float32)
    out[...] = jax.nn.sigmoid(r)


def _tc_rating(ue2, ue1, ucb, acc2, acc1, items_cnt,
               f1w, f1b, f2w, f2b, f3w, f3b, f4w, f4b):
    full = lambda j: (0, 0)
    return pl.pallas_call(
        _tc_rating_body,
        grid=(IGRID,),
        in_specs=[
            pl.BlockSpec((BATCH, D), full),
            pl.BlockSpec((BATCH, D), full),
            pl.BlockSpec((BATCH, 1), full),
            pl.BlockSpec((IBLK, D), lambda j: (IGRID + j, 0)),
            pl.BlockSpec((IBLK, D), lambda j: (IGRID + j, 0)),
            pl.BlockSpec((IBLK, 1), lambda j: (j, 0)),
            pl.BlockSpec((D, 1), full),
            pl.BlockSpec((1, 1), full),
            pl.BlockSpec((D, 1), full),
            pl.BlockSpec((1, 1), full),
            pl.BlockSpec((D, 1), full),
            pl.BlockSpec((1, 1), full),
            pl.BlockSpec((D, 1), full),
            pl.BlockSpec((1, 1), full),
        ],
        out_specs=pl.BlockSpec((BATCH, IBLK), lambda j: (0, j)),
        out_shape=jax.ShapeDtypeStruct((BATCH, NUM_ITEMS), jnp.float32),
    )(ue2, ue1, ucb, acc2, acc1, items_cnt,
      f1w, f1b, f2w, f2b, f3w, f3b, f4w, f4b)


def _pad_table(emb_u, emb_i):
    zpad = jnp.zeros((PAD_OFF, D), jnp.float32)
    return jnp.concatenate([emb_u, zpad, emb_i, zpad], axis=0)


def _propagate_sc(edge_index, vals, x_pad):
    pad = E_PAD - E
    rows = jnp.pad(edge_index[0], (0, pad), constant_values=-1)
    cols = jnp.pad(edge_index[1], (0, pad), constant_values=0)
    v = jnp.pad(vals, (0, pad), constant_values=0.0)
    segc, segr, segv, cnt = _sc_route(cols, rows, v)
    tab = x_pad
    acc = x_pad
    for _ in range(N_LAYERS):
        tab, acc = _sc_layer(segc, segr, segv, cnt, tab, acc)
    return acc


def kernel(emb_u1, emb_i1, emb_u2, emb_i2, fc1_w, fc1_b, fc2_w, fc2_b,
           fc3_w, fc3_b, fc4_w, fc4_b, vals1, vals2, users_cnt, items_cnt,
           users, edge_index1, edge_index2):
    x2 = _pad_table(emb_u2, emb_i2)
    x1 = _pad_table(emb_u1, emb_i1)
    acc2 = _propagate_sc(edge_index2, vals2, x2)
    acc1 = _propagate_sc(edge_index1, vals1, x1)

    ue2, ue1, ucb = _sc_batch_gather(users, acc2, acc1,
                                     users_cnt.reshape(NUM_USERS))
    rating = _tc_rating(ue2, ue1, ucb.reshape(BATCH, 1), acc2, acc1,
                        items_cnt,
                        fc1_w, fc1_b.reshape(1, 1), fc2_w, fc2_b.reshape(1, 1),
                        fc3_w, fc3_b.reshape(1, 1), fc4_w, fc4_b.reshape(1, 1))
    return rating


# speedup vs baseline: 12.6249x; 1.0082x over previous
import functools

import jax
import jax.numpy as jnp
from jax import lax
from jax.experimental import pallas as pl
from jax.experimental.pallas import tpu as pltpu
from jax.experimental.pallas import tpu_sc as plsc

NUM_USERS = 50000
NUM_ITEMS = 50000
N_NODES = NUM_USERS + NUM_ITEMS
D = 32
N_LAYERS = 3
E = 1600000
LAM1 = 0.9
LAM2 = 0.9
BATCH = 1024

HALF = 50176
NP = 2 * HALF
PAD_OFF = HALF - NUM_USERS

NSC = 2
NTS = 16
NW = NSC * NTS
CAP = 50176
E_PAD = NW * CAP
SEG = NW * CAP
BLK = 256
NBR = CAP // BLK
CHUNK = 128
NCH = BLK // CHUNK
RPT = HALF // NTS
RCH = 112
NRCH = RPT // RCH
STG = 544

_mesh = plsc.VectorSubcoreMesh(core_axis_name="c", subcore_axis_name="s")
_I32 = jnp.int32


def _route_body(col_h, row_h, val_h, segc_h, segr_h, segv_h, cnt_h,
                colb, rowb, valb, sc0, sr0, sv0, sc1, sr1, sv1, cntv, sem_r):
    c = lax.axis_index("c")
    s = lax.axis_index("s")
    j = s * NSC + c
    ebase = j * CAP
    trash = NUM_USERS + j // 2

    def issue_in(p, base):
        pltpu.async_copy(col_h.at[pl.ds(base, BLK)], colb.at[p], sem_r)
        pltpu.async_copy(row_h.at[pl.ds(base, BLK)], rowb.at[p], sem_r)
        pltpu.async_copy(val_h.at[pl.ds(base, BLK)], valb.at[p], sem_r)

    def wait_in(p):
        pltpu.make_async_copy(col_h.at[pl.ds(0, BLK)], colb.at[p],
                              sem_r).wait()
        pltpu.make_async_copy(row_h.at[pl.ds(0, BLK)], rowb.at[p],
                              sem_r).wait()
        pltpu.make_async_copy(val_h.at[pl.ds(0, BLK)], valb.at[p],
                              sem_r).wait()

    issue_in(0, ebase)

    def blk_fn(b, carry):
        f0, f1, bc0, bc1 = carry
        p = lax.rem(b, 2)
        wait_in(p)

        @pl.when(b < NBR - 1)
        def _():
            issue_in(1 - p, ebase + (b + 1) * BLK)

        def grp(g, cr):
            f0, f1 = cr
            o = g * 16
            cg = colb[p, pl.ds(o, 16)]
            rg = rowb[p, pl.ds(o, 16)]
            vg = valb[p, pl.ds(o, 16)]
            pc = cg + jnp.where(cg >= NUM_USERS, PAD_OFF, 0)
            m0 = (rg >= 0) & (rg < NUM_USERS)
            m1 = rg >= NUM_USERS
            cs0 = plsc.cumsum(m0.astype(_I32))
            pos0 = f0 + cs0 - 1
            plsc.store_scatter(sc0, [pos0], pc, mask=m0)
            plsc.store_scatter(sr0, [pos0], rg, mask=m0)
            plsc.store_scatter(sv0, [pos0], vg, mask=m0)
            cs1 = plsc.cumsum(m1.astype(_I32))
            pos1 = f1 + cs1 - 1
            plsc.store_scatter(sc1, [pos1], pc, mask=m1)
            plsc.store_scatter(sr1, [pos1], rg - NUM_USERS, mask=m1)
            plsc.store_scatter(sv1, [pos1], vg, mask=m1)
            n0 = jnp.max(cs0)
            n1 = jnp.max(cs1)
            return (f0 + n0, f1 + n1)
        f0, f1 = lax.fori_loop(0, BLK // 16, grp, (f0, f1))

        def flush(stc, str_, stv, seg_off, f, bc, do):
            @pl.when(do)
            def _():
                pltpu.sync_copy(stc.at[pl.ds(0, BLK)],
                                segc_h.at[pl.ds(seg_off + bc * BLK, BLK)])
                pltpu.sync_copy(str_.at[pl.ds(0, BLK)],
                                segr_h.at[pl.ds(seg_off + bc * BLK, BLK)])
                pltpu.sync_copy(stv.at[pl.ds(0, BLK)],
                                segv_h.at[pl.ds(seg_off + bc * BLK, BLK)])

                def mv(g, _):
                    o = g * 16
                    stc[pl.ds(o, 16)] = stc[pl.ds(BLK + o, 16)]
                    str_[pl.ds(o, 16)] = str_[pl.ds(BLK + o, 16)]
                    stv[pl.ds(o, 16)] = stv[pl.ds(BLK + o, 16)]
                    return _
                lax.fori_loop(0, BLK // 16, mv, None)
            return (jnp.where(do, f - BLK, f), jnp.where(do, bc + 1, bc))

        f0, bc0 = flush(sc0, sr0, sv0, (j * CAP), f0, bc0, f0 >= BLK)
        f1, bc1 = flush(sc1, sr1, sv1, (SEG + j * CAP), f1, bc1, f1 >= BLK)
        return (f0, f1, bc0, bc1)

    z = jnp.int32(0)
    f0, f1, bc0, bc1 = lax.fori_loop(0, NBR, blk_fn, (z, z, z, z))

    def tail(stc, str_, stv, seg_off, f, bc):
        def pg(g, _):
            o = g * 16
            lane = o + lax.iota(jnp.int32, 16)
            keep = lane < f
            stc[pl.ds(o, 16)] = jnp.where(keep, stc[pl.ds(o, 16)], 0)
            str_[pl.ds(o, 16)] = jnp.where(keep, str_[pl.ds(o, 16)], trash)
            stv[pl.ds(o, 16)] = jnp.where(keep, stv[pl.ds(o, 16)], 0.0)
            return _
        lax.fori_loop(0, BLK // 16, pg, None)

        @pl.when(f > 0)
        def _():
            pltpu.sync_copy(stc.at[pl.ds(0, BLK)],
                            segc_h.at[pl.ds(seg_off + bc * BLK, BLK)])
            pltpu.sync_copy(str_.at[pl.ds(0, BLK)],
                            segr_h.at[pl.ds(seg_off + bc * BLK, BLK)])
            pltpu.sync_copy(stv.at[pl.ds(0, BLK)],
                            segv_h.at[pl.ds(seg_off + bc * BLK, BLK)])
        return jnp.where(f > 0, bc + 1, bc)

    bc0 = tail(sc0, sr0, sv0, (j * CAP), f0, bc0)
    bc1 = tail(sc1, sr1, sv1, (SEG + j * CAP), f1, bc1)

    cntv[pl.ds(0, 16)] = jnp.full((16,), bc0, _I32)
    pltpu.sync_copy(cntv, cnt_h.at[0, j])
    cntv[pl.ds(0, 16)] = jnp.full((16,), bc1, _I32)
    pltpu.sync_copy(cntv, cnt_h.at[1, j])


@functools.partial(
    pl.kernel,
    out_type=(jax.ShapeDtypeStruct((2 * SEG,), _I32),
              jax.ShapeDtypeStruct((2 * SEG,), _I32),
              jax.ShapeDtypeStruct((2 * SEG,), jnp.float32),
              jax.ShapeDtypeStruct((2, NW, 16), _I32)),
    mesh=_mesh,
    scratch_types=[
        pltpu.VMEM((2, BLK), _I32),
        pltpu.VMEM((2, BLK), _I32),
        pltpu.VMEM((2, BLK), jnp.float32),
        pltpu.VMEM((STG,), _I32),
        pltpu.VMEM((STG,), _I32),
        pltpu.VMEM((STG,), jnp.float32),
        pltpu.VMEM((STG,), _I32),
        pltpu.VMEM((STG,), _I32),
        pltpu.VMEM((STG,), jnp.float32),
        pltpu.VMEM((16,), _I32),
        pltpu.SemaphoreType.DMA,
    ],
    compiler_params=pltpu.CompilerParams(use_tc_tiling_on_sc=False,
                                         needs_layout_passes=False),
)
def _sc_route(col_h, row_h, val_h, *rest):
    _route_body(col_h, row_h, val_h, *rest)


def _sc_layer_body(segc_h, segr_h, segv_h, cnt_h, tab_h, accin_h,
                   newtab_h, accout_h,
                   col_v, val_v, idx_v, src_v, zb_v, ab_v, cnt_v, acc_sh,
                   sem_s, sem_g, sem_w):
    c = lax.axis_index("c")
    s = lax.axis_index("s")

    def zrow(i, _):
        zb_v[i, pl.ds(0, 16)] = jnp.zeros((16,), jnp.float32)
        zb_v[i, pl.ds(16, 16)] = jnp.zeros((16,), jnp.float32)
        return _
    lax.fori_loop(0, RCH, zrow, None)
    for k in range(NRCH):
        pltpu.sync_copy(zb_v, acc_sh.at[pl.ds(s * RPT + k * RCH, RCH)])
    plsc.subcore_barrier()

    pltpu.sync_copy(cnt_h.at[c, 2 * s], cnt_v)
    nb0 = jnp.max(cnt_v[...])
    pltpu.sync_copy(cnt_h.at[c, 2 * s + 1], cnt_v)
    nb1 = jnp.max(cnt_v[...])
    nb = nb0 + nb1
    hoff = c * SEG

    def base(b):
        return hoff + jnp.where(
            b < nb0, (2 * s) * CAP + b * BLK,
            (2 * s + 1) * CAP + (b - nb0) * BLK)

    def issue_smalls(p, base_):
        pltpu.async_copy(segc_h.at[pl.ds(base_, BLK)], col_v.at[p], sem_s)
        for k in range(NCH):
            pltpu.async_copy(segr_h.at[pl.ds(base_ + k * CHUNK, CHUNK)],
                             idx_v.at[p, k], sem_s)
        pltpu.async_copy(segv_h.at[pl.ds(base_, BLK)], val_v.at[p], sem_s)

    def wait_smalls(p):
        pltpu.make_async_copy(segc_h.at[pl.ds(0, BLK)], col_v.at[p],
                              sem_s).wait()
        for k in range(NCH):
            pltpu.make_async_copy(segr_h.at[pl.ds(0, CHUNK)],
                                  idx_v.at[p, k], sem_s).wait()
        pltpu.make_async_copy(segv_h.at[pl.ds(0, BLK)], val_v.at[p],
                              sem_s).wait()

    def issue_gather(p):
        for k in range(NCH):
            pltpu.async_copy(
                tab_h.at[col_v.at[p, pl.ds(k * CHUNK, CHUNK)]],
                src_v.at[p, pl.ds(k * CHUNK, CHUNK)], sem_g)

    def wait_gather(p):
        for k in range(NCH):
            pltpu.make_async_copy(
                tab_h.at[col_v.at[p, pl.ds(k * CHUNK, CHUNK)]],
                src_v.at[p, pl.ds(k * CHUNK, CHUNK)], sem_g).wait()

    def issue_scatter(p):
        for k in range(NCH):
            pltpu.async_copy(src_v.at[p, pl.ds(k * CHUNK, CHUNK)],
                             acc_sh.at[idx_v.at[p, k]], sem_w, add=True)

    def wait_scatter(p):
        for k in range(NCH):
            pltpu.make_async_copy(src_v.at[p, pl.ds(k * CHUNK, CHUNK)],
                                  acc_sh.at[idx_v.at[p, k]], sem_w).wait()

    def compute(p):
        def grp(g, _):
            o = g * 16
            vv = val_v[p, pl.ds(o, 16)]
            for e in range(16):
                eg = o + e
                bc = lax.gather(
                    vv, jnp.full((16, 1), e, _I32),
                    lax.GatherDimensionNumbers(offset_dims=(),
                                               collapsed_slice_dims=(0,),
                                               start_index_map=(0,)),
                    (1,), mode=lax.GatherScatterMode.PROMISE_IN_BOUNDS)
                src_v[p, eg, pl.ds(0, 16)] = src_v[p, eg, pl.ds(0, 16)] * bc
                src_v[p, eg, pl.ds(16, 16)] = src_v[p, eg, pl.ds(16, 16)] * bc
            return _
        lax.fori_loop(0, BLK // 16, grp, None)

    @pl.when(nb > 0)
    def _():
        issue_smalls(0, base(0))
        wait_smalls(0)
        issue_gather(0)

    @pl.when(nb > 1)
    def _():
        issue_smalls(1, base(1))

    def blk(b, _):
        p = lax.rem(b, 2)
        q = 1 - p

        @pl.when(b < nb - 1)
        def _():
            wait_smalls(q)

            @pl.when(b > 0)
            def _():
                wait_scatter(q)
            issue_gather(q)

        wait_gather(p)
        compute(p)
        issue_scatter(p)

        @pl.when(b < nb - 2)
        def _():
            issue_smalls(p, base(b + 2))
        return _
    lax.fori_loop(0, nb, blk, None)

    @pl.when(nb > 1)
    def _():
        wait_scatter(lax.rem(nb, 2))

    @pl.when(nb > 0)
    def _():
        wait_scatter(lax.rem(nb + 1, 2))
    plsc.subcore_barrier()

    for k in range(NRCH):
        r0 = s * RPT + k * RCH
        g0 = c * HALF + r0
        pltpu.sync_copy(acc_sh.at[pl.ds(r0, RCH)], zb_v)
        pltpu.sync_copy(accin_h.at[pl.ds(g0, RCH)], ab_v)

        def acc_row(i, _):
            ab_v[i, pl.ds(0, 16)] = ab_v[i, pl.ds(0, 16)] + zb_v[i, pl.ds(0, 16)]
            ab_v[i, pl.ds(16, 16)] = (ab_v[i, pl.ds(16, 16)]
                                      + zb_v[i, pl.ds(16, 16)])
            return _
        lax.fori_loop(0, RCH, acc_row, None)
        pltpu.sync_copy(zb_v, newtab_h.at[pl.ds(g0, RCH)])
        pltpu.sync_copy(ab_v, accout_h.at[pl.ds(g0, RCH)])


@functools.partial(
    pl.kernel,
    out_type=(jax.ShapeDtypeStruct((NP, D), jnp.float32),
              jax.ShapeDtypeStruct((NP, D), jnp.float32)),
    mesh=_mesh,
    scratch_types=[
        pltpu.VMEM((2, BLK), _I32),
        pltpu.VMEM((2, BLK), jnp.float32),
        pltpu.VMEM((2, NCH, CHUNK), _I32),
        pltpu.VMEM((2, BLK, D), jnp.float32),
        pltpu.VMEM((RCH, D), jnp.float32),
        pltpu.VMEM((RCH, D), jnp.float32),
        pltpu.VMEM((16,), _I32),
        pltpu.VMEM_SHARED((HALF, D), jnp.float32),
        pltpu.SemaphoreType.DMA,
        pltpu.SemaphoreType.DMA,
        pltpu.SemaphoreType.DMA,
    ],
    compiler_params=pltpu.CompilerParams(use_tc_tiling_on_sc=False,
                                         needs_layout_passes=False),
)
def _sc_layer(segc_h, segr_h, segv_h, cnt_h, tab_h, accin_h, *rest):
    _sc_layer_body(segc_h, segr_h, segv_h, cnt_h, tab_h, accin_h, *rest)


UPT = BATCH // NW


@functools.partial(
    pl.kernel,
    out_type=(jax.ShapeDtypeStruct((BATCH, D), jnp.float32),
              jax.ShapeDtypeStruct((BATCH, D), jnp.float32),
              jax.ShapeDtypeStruct((BATCH,), jnp.float32)),
    mesh=_mesh,
    scratch_types=[
        pltpu.VMEM((UPT,), _I32),
        pltpu.VMEM((UPT, D), jnp.float32),
        pltpu.VMEM((UPT, D), jnp.float32),
        pltpu.VMEM((UPT,), jnp.float32),
        pltpu.SemaphoreType.DMA,
    ],
    compiler_params=pltpu.CompilerParams(use_tc_tiling_on_sc=False),
)
def _sc_batch_gather(users_h, acc2_h, acc1_h, ucnt_h, ue2_h, ue1_h, ucb_h,
                     uidx_v, u2_v, u1_v, uc_v, sem):
    c = lax.axis_index("c")
    s = lax.axis_index("s")
    wid = s * NSC + c
    base = wid * UPT
    pltpu.sync_copy(users_h.at[pl.ds(base, UPT)], uidx_v)
    pltpu.async_copy(acc2_h.at[uidx_v], u2_v, sem).wait()
    pltpu.async_copy(acc1_h.at[uidx_v], u1_v, sem).wait()
    pltpu.async_copy(ucnt_h.at[uidx_v], uc_v, sem).wait()
    pltpu.sync_copy(u2_v, ue2_h.at[pl.ds(base, UPT)])
    pltpu.sync_copy(u1_v, ue1_h.at[pl.ds(base, UPT)])
    pltpu.sync_copy(uc_v, ucb_h.at[pl.ds(base, UPT)])


IBLK = 512
IGRID = 98


def _tc_rating_body(ue2, ue1, ucb, i2, i1, icnt,
                    f1w, f1b, f2w, f2b, f3w, f3b, f4w, f4b, out):
    au = ue2[...] * 0.25
    au2 = ue1[...] * 0.25
    z1 = (jax.lax.dot_general(au, f1w[...], (((1,), (0,)), ((), ())),
                              preferred_element_type=jnp.float32) + f1b[...]
          + jax.lax.dot_general(au2, f2w[...], (((1,), (0,)), ((), ())),
                                preferred_element_type=jnp.float32) + f2b[...])
    w1 = jax.nn.sigmoid(z1)
    w1 = ucb[...] * (1.0 - LAM1) + w1 * LAM1
    u = au * w1 + au2 * (1.0 - w1)

    ai = i2[...] * 0.25
    ai2 = i1[...] * 0.25
    z2 = (jax.lax.dot_general(ai, f3w[...], (((1,), (0,)), ((), ())),
                              preferred_element_type=jnp.float32) + f3b[...]
          + jax.lax.dot_general(ai2, f4w[...], (((1,), (0,)), ((), ())),
                                preferred_element_type=jnp.float32) + f4b[...])
    w2 = jax.nn.sigmoid(z2)
    w2 = icnt[...] * (1.0 - LAM2) + w2 * LAM2
    fi = ai * w2 + ai2 * (1.0 - w2)

    r = jax.lax.dot_general(u, fi, (((1,), (1,)), ((), ())),
                            preferred_element_type=jnp.float32)
    out[...] = jax.nn.sigmoid(r)


def _tc_rating(ue2, ue1, ucb, acc2, acc1, items_cnt,
               f1w, f1b, f2w, f2b, f3w, f3b, f4w, f4b):
    full = lambda j: (0, 0)
    return pl.pallas_call(
        _tc_rating_body,
        grid=(IGRID,),
        in_specs=[
            pl.BlockSpec((BATCH, D), full),
            pl.BlockSpec((BATCH, D), full),
            pl.BlockSpec((BATCH, 1), full),
            pl.BlockSpec((IBLK, D), lambda j: (IGRID + j, 0)),
            pl.BlockSpec((IBLK, D), lambda j: (IGRID + j, 0)),
            pl.BlockSpec((IBLK, 1), lambda j: (j, 0)),
            pl.BlockSpec((D, 1), full),
            pl.BlockSpec((1, 1), full),
            pl.BlockSpec((D, 1), full),
            pl.BlockSpec((1, 1), full),
            pl.BlockSpec((D, 1), full),
            pl.BlockSpec((1, 1), full),
            pl.BlockSpec((D, 1), full),
            pl.BlockSpec((1, 1), full),
        ],
        out_specs=pl.BlockSpec((BATCH, IBLK), lambda j: (0, j)),
        out_shape=jax.ShapeDtypeStruct((BATCH, NUM_ITEMS), jnp.float32),
    )(ue2, ue1, ucb, acc2, acc1, items_cnt,
      f1w, f1b, f2w, f2b, f3w, f3b, f4w, f4b)


def _pad_table(emb_u, emb_i):
    zpad = jnp.zeros((PAD_OFF, D), jnp.float32)
    return jnp.concatenate([emb_u, zpad, emb_i, zpad], axis=0)


def _propagate_sc(edge_index, vals, x_pad):
    pad = E_PAD - E
    rows = jnp.pad(edge_index[0], (0, pad), constant_values=-1)
    cols = jnp.pad(edge_index[1], (0, pad), constant_values=0)
    v = jnp.pad(vals, (0, pad), constant_values=0.0)
    segc, segr, segv, cnt = _sc_route(cols, rows, v)
    tab = x_pad
    acc = x_pad
    for _ in range(N_LAYERS):
        tab, acc = _sc_layer(segc, segr, segv, cnt, tab, acc)
    return acc


def kernel(emb_u1, emb_i1, emb_u2, emb_i2, fc1_w, fc1_b, fc2_w, fc2_b,
           fc3_w, fc3_b, fc4_w, fc4_b, vals1, vals2, users_cnt, items_cnt,
           users, edge_index1, edge_index2):
    x2 = _pad_table(emb_u2, emb_i2)
    x1 = _pad_table(emb_u1, emb_i1)
    acc2 = _propagate_sc(edge_index2, vals2, x2)
    acc1 = _propagate_sc(edge_index1, vals1, x1)

    ue2, ue1, ucb = _sc_batch_gather(users, acc2, acc1,
                                     users_cnt.reshape(NUM_USERS))
    rating = _tc_rating(ue2, ue1, ucb.reshape(BATCH, 1), acc2, acc1,
                        items_cnt,
                        fc1_w, fc1_b.reshape(1, 1), fc2_w, fc2_b.reshape(1, 1),
                        fc3_w, fc3_b.reshape(1, 1), fc4_w, fc4_b.reshape(1, 1))
    return rating
